# baseline (speedup 1.0000x reference)
import functools

import jax
import jax.numpy as jnp
from jax.experimental import pallas as pl

DIM_STATE_CONT = 23


def _mlp_kernel(x_ref, w1_ref, b1_ref, wd_ref, bd_ref, wm_ref, bm_ref,
                ws_ref, bs_ref, disc_ref, mean_ref, std_ref):
    h = jnp.dot(x_ref[...], w1_ref[...],
                preferred_element_type=jnp.float32,
                precision=jax.lax.Precision.HIGHEST) + b1_ref[...]
    h = jnp.where(h >= 0, h, 0.01 * h)
    disc_ref[...] = jnp.dot(h, wd_ref[...],
                            preferred_element_type=jnp.float32,
                            precision=jax.lax.Precision.HIGHEST) + bd_ref[...]
    mean = jnp.dot(h, wm_ref[...],
                   preferred_element_type=jnp.float32,
                   precision=jax.lax.Precision.HIGHEST) + bm_ref[...]
    mean_ref[...] = jnp.clip(mean, -1.0, 1.0)
    std = jnp.dot(h, ws_ref[...],
                  preferred_element_type=jnp.float32,
                  precision=jax.lax.Precision.HIGHEST) + bs_ref[...]
    std_ref[...] = jnp.clip(std, 0.0, 1.0)


@functools.partial(jax.jit, static_argnames=("block_m",))
def _run(x, W1, b1, W_disc, b_disc, W_cont, b_cont, block_m=1024):
    batch, dim_in = x.shape
    dim_h = W1.shape[1]
    dim_disc = W_disc.shape[1]
    nc = DIM_STATE_CONT

    W_mean = W_cont[:, :nc]
    W_std = W_cont[:, nc:]
    b_mean = b_cont[:nc].reshape(1, nc)
    b_std = b_cont[nc:].reshape(1, nc)
    b1r = b1.reshape(1, dim_h)
    b_disc_r = b_disc.reshape(1, dim_disc)

    grid = (batch // block_m,)
    row_spec = lambda w: pl.BlockSpec((block_m, w), lambda i: (i, 0))
    full_spec = lambda r, c: pl.BlockSpec((r, c), lambda i: (0, 0))

    return pl.pallas_call(
        _mlp_kernel,
        grid=grid,
        in_specs=[
            row_spec(dim_in),
            full_spec(dim_in, dim_h),
            full_spec(1, dim_h),
            full_spec(dim_h, dim_disc),
            full_spec(1, dim_disc),
            full_spec(dim_h, nc),
            full_spec(1, nc),
            full_spec(dim_h, nc),
            full_spec(1, nc),
        ],
        out_specs=[
            row_spec(dim_disc),
            row_spec(nc),
            row_spec(nc),
        ],
        out_shape=[
            jax.ShapeDtypeStruct((batch, dim_disc), jnp.float32),
            jax.ShapeDtypeStruct((batch, nc), jnp.float32),
            jax.ShapeDtypeStruct((batch, nc), jnp.float32),
        ],
    )(x, W1, b1r, W_disc, b_disc_r, W_mean, b_mean, W_std, b_std)


def kernel(x, W1, b1, W_disc, b_disc, W_cont, b_cont):
    disc, mean, std = _run(x, W1, b1, W_disc, b_disc, W_cont, b_cont)
    return (disc, mean, std)


# speedup vs baseline: 1.7589x; 1.7589x over previous
import functools

import jax
import jax.numpy as jnp
from jax.experimental import pallas as pl

DIM_STATE_CONT = 23


def _mlp_kernel(x_ref, w1_ref, b1_ref, wd_ref, bd_ref, wm_ref, bm_ref,
                ws_ref, bs_ref, disc_ref, mean_ref, std_ref):
    h = jnp.dot(x_ref[...], w1_ref[...],
                preferred_element_type=jnp.float32) + b1_ref[...]
    h = jnp.where(h >= 0, h, 0.01 * h)
    disc_ref[...] = jnp.dot(h, wd_ref[...],
                            preferred_element_type=jnp.float32) + bd_ref[...]
    mean = jnp.dot(h, wm_ref[...],
                   preferred_element_type=jnp.float32) + bm_ref[...]
    mean_ref[...] = jnp.clip(mean, -1.0, 1.0)
    std = jnp.dot(h, ws_ref[...],
                  preferred_element_type=jnp.float32) + bs_ref[...]
    std_ref[...] = jnp.clip(std, 0.0, 1.0)


@functools.partial(jax.jit, static_argnames=("block_m",))
def _run(x, W1, b1, W_disc, b_disc, W_cont, b_cont, block_m=1024):
    batch, dim_in = x.shape
    dim_h = W1.shape[1]
    dim_disc = W_disc.shape[1]
    nc = DIM_STATE_CONT

    W_mean = W_cont[:, :nc]
    W_std = W_cont[:, nc:]
    b_mean = b_cont[:nc].reshape(1, nc)
    b_std = b_cont[nc:].reshape(1, nc)
    b1r = b1.reshape(1, dim_h)
    b_disc_r = b_disc.reshape(1, dim_disc)

    grid = (batch // block_m,)
    row_spec = lambda w: pl.BlockSpec((block_m, w), lambda i: (i, 0))
    full_spec = lambda r, c: pl.BlockSpec((r, c), lambda i: (0, 0))

    return pl.pallas_call(
        _mlp_kernel,
        grid=grid,
        in_specs=[
            row_spec(dim_in),
            full_spec(dim_in, dim_h),
            full_spec(1, dim_h),
            full_spec(dim_h, dim_disc),
            full_spec(1, dim_disc),
            full_spec(dim_h, nc),
            full_spec(1, nc),
            full_spec(dim_h, nc),
            full_spec(1, nc),
        ],
        out_specs=[
            row_spec(dim_disc),
            row_spec(nc),
            row_spec(nc),
        ],
        out_shape=[
            jax.ShapeDtypeStruct((batch, dim_disc), jnp.float32),
            jax.ShapeDtypeStruct((batch, nc), jnp.float32),
            jax.ShapeDtypeStruct((batch, nc), jnp.float32),
        ],
    )(x, W1, b1r, W_disc, b_disc_r, W_mean, b_mean, W_std, b_std)


def kernel(x, W1, b1, W_disc, b_disc, W_cont, b_cont):
    disc, mean, std = _run(x, W1, b1, W_disc, b_disc, W_cont, b_cont)
    return (disc, mean, std)


# speedup vs baseline: 1.9479x; 1.1075x over previous
import functools

import jax
import jax.numpy as jnp
from jax.experimental import pallas as pl

DIM_STATE_CONT = 23


def _mlp_kernel(x_ref, w1_ref, b1_ref, wd_ref, bd_ref, wm_ref, bm_ref,
                ws_ref, bs_ref, disc_ref, mean_ref, std_ref):
    h = jnp.dot(x_ref[...], w1_ref[...],
                preferred_element_type=jnp.float32) + b1_ref[...]
    h = jnp.where(h >= 0, h, 0.01 * h)
    disc_ref[...] = jnp.dot(h, wd_ref[...],
                            preferred_element_type=jnp.float32) + bd_ref[...]
    mean = jnp.dot(h, wm_ref[...],
                   preferred_element_type=jnp.float32) + bm_ref[...]
    mean_ref[...] = jnp.clip(mean, -1.0, 1.0)
    std = jnp.dot(h, ws_ref[...],
                  preferred_element_type=jnp.float32) + bs_ref[...]
    std_ref[...] = jnp.clip(std, 0.0, 1.0)


@functools.partial(jax.jit, static_argnames=("block_m",))
def _run(x, W1, b1, W_disc, b_disc, W_cont, b_cont, block_m=1024):
    batch, dim_in = x.shape
    dim_h = W1.shape[1]
    dim_disc = W_disc.shape[1]
    nc = DIM_STATE_CONT

    W_mean = W_cont[:, :nc]
    W_std = W_cont[:, nc:]
    b_mean = b_cont[:nc].reshape(1, nc)
    b_std = b_cont[nc:].reshape(1, nc)
    b1r = b1.reshape(1, dim_h)
    b_disc_r = b_disc.reshape(1, dim_disc)

    grid = (batch // block_m,)
    row_spec = lambda w: pl.BlockSpec((block_m, w), lambda i: (i, 0))
    full_spec = lambda r, c: pl.BlockSpec((r, c), lambda i: (0, 0))

    return pl.pallas_call(
        _mlp_kernel,
        grid=grid,
        in_specs=[
            row_spec(dim_in),
            full_spec(dim_in, dim_h),
            full_spec(1, dim_h),
            full_spec(dim_h, dim_disc),
            full_spec(1, dim_disc),
            full_spec(dim_h, nc),
            full_spec(1, nc),
            full_spec(dim_h, nc),
            full_spec(1, nc),
        ],
        out_specs=[
            row_spec(dim_disc),
            row_spec(nc),
            row_spec(nc),
        ],
        out_shape=[
            jax.ShapeDtypeStruct((batch, dim_disc), jnp.float32),
            jax.ShapeDtypeStruct((batch, nc), jnp.float32),
            jax.ShapeDtypeStruct((batch, nc), jnp.float32),
        ],
    )(x, W1, b1r, W_disc, b_disc_r, W_mean, b_mean, W_std, b_std)


import os
_BM = int(os.environ.get("KBM", "1024"))


def kernel(x, W1, b1, W_disc, b_disc, W_cont, b_cont):
    disc, mean, std = _run(x, W1, b1, W_disc, b_disc, W_cont, b_cont,
                           block_m=_BM)
    return (disc, mean, std)
